# Initial kernel scaffold; baseline (speedup 1.0000x reference)
import functools

import jax
import jax.numpy as jnp
from jax import lax
from jax.experimental import pallas as pl
from jax.experimental.pallas import tpu as pltpu
from jax.experimental.pallas import tpu_sc as plsc

N = 10000
E = 320000
D = 128
NPAD = 10240
NT = 32
CH = 128
NCH = 80
EPAD = NT * NCH * CH
ROWS_PER_TILE = NPAD // 16

_mesh = plsc.VectorSubcoreMesh(core_axis_name="c", subcore_axis_name="s")


def _fill_loop(ref, nrows, width, value):
    def body(i, _):
        for j in range(width // 16):
            ref[i, pl.ds(j * 16, 16)] = jnp.full((16,), value, jnp.float32)
        return 0
    lax.fori_loop(0, nrows, body, 0)


def _make_edge_agg(feat):
    @functools.partial(
        pl.kernel,
        out_type=jax.ShapeDtypeStruct((2, NPAD, feat), jnp.float32),
        mesh=_mesh,
        scratch_types=[
            pltpu.VMEM((NCH, CH), jnp.int32),
            pltpu.VMEM((NCH, CH), jnp.int32),
            pltpu.VMEM((CH, feat), jnp.float32),
            pltpu.VMEM_SHARED((NPAD, feat), jnp.float32),
            pltpu.SemaphoreType.DMA,
        ],
    )
    def agg(hp_hbm, srcT_hbm, dstT_hbm, out_hbm, src_v, dst_v, rows_v, acc_sh, sem):
        cid = lax.axis_index("c")
        sid = lax.axis_index("s")
        wid = sid * 2 + cid
        pltpu.sync_copy(srcT_hbm.at[wid], src_v)
        pltpu.sync_copy(dstT_hbm.at[wid], dst_v)
        _fill_loop(rows_v, CH, feat, 0.0)
        for k in range(ROWS_PER_TILE // CH):
            pltpu.sync_copy(rows_v, acc_sh.at[pl.ds(sid * ROWS_PER_TILE + k * CH, CH)])
        plsc.subcore_barrier()

        def step(c, _):
            pltpu.async_copy(hp_hbm.at[src_v.at[c]], rows_v, sem).wait()
            pltpu.sync_copy(rows_v, acc_sh.at[dst_v.at[c]], add=True)
            return 0
        lax.fori_loop(0, NCH, step, 0)

        plsc.subcore_barrier()
        for k in range(ROWS_PER_TILE // CH):
            sl = pl.ds(sid * ROWS_PER_TILE + k * CH, CH)
            pltpu.sync_copy(acc_sh.at[sl], rows_v)
            pltpu.sync_copy(rows_v, out_hbm.at[cid, sl])
    return agg


_edge_agg_128 = _make_edge_agg(128)
_edge_agg_16 = _make_edge_agg(16)


@functools.partial(
    pl.kernel,
    out_type=jax.ShapeDtypeStruct((2, NPAD, 16), jnp.float32),
    mesh=_mesh,
    scratch_types=[
        pltpu.VMEM((NCH, CH), jnp.int32),
        pltpu.VMEM((CH, 16), jnp.float32),
        pltpu.VMEM((CH, 16), jnp.float32),
        pltpu.VMEM_SHARED((NPAD, 16), jnp.float32),
    ],
)
def _deg_count(dstT_hbm, out_hbm, dst_v, ones_v, buf_v, acc_sh):
    cid = lax.axis_index("c")
    sid = lax.axis_index("s")
    wid = sid * 2 + cid
    pltpu.sync_copy(dstT_hbm.at[wid], dst_v)
    _fill_loop(ones_v, CH, 16, 1.0)
    _fill_loop(buf_v, CH, 16, 0.0)
    for k in range(ROWS_PER_TILE // CH):
        pltpu.sync_copy(buf_v, acc_sh.at[pl.ds(sid * ROWS_PER_TILE + k * CH, CH)])
    plsc.subcore_barrier()

    def step(c, _):
        pltpu.sync_copy(ones_v, acc_sh.at[dst_v.at[c]], add=True)
        return 0
    lax.fori_loop(0, NCH, step, 0)

    plsc.subcore_barrier()
    for k in range(ROWS_PER_TILE // CH):
        sl = pl.ds(sid * ROWS_PER_TILE + k * CH, CH)
        pltpu.sync_copy(acc_sh.at[sl], buf_v)
        pltpu.sync_copy(buf_v, out_hbm.at[cid, sl])


_BR = 1000
_GRID = N // _BR


def _tc_a_body(deg_ref, x_ref, dinv_ref, hp_ref):
    deg = deg_ref[0, :, 0] + deg_ref[1, :, 0] + 1.0
    dinv = lax.rsqrt(deg)[:, None]
    dinv_ref[...] = dinv
    hp_ref[...] = x_ref[...] * dinv


def _tc_a(deg2, x):
    return pl.pallas_call(
        _tc_a_body,
        grid=(_GRID,),
        in_specs=[
            pl.BlockSpec((2, _BR, 16), lambda i: (0, i, 0)),
            pl.BlockSpec((_BR, D), lambda i: (i, 0)),
        ],
        out_specs=[
            pl.BlockSpec((_BR, 1), lambda i: (i, 0)),
            pl.BlockSpec((_BR, D), lambda i: (i, 0)),
        ],
        out_shape=[
            jax.ShapeDtypeStruct((N, 1), jnp.float32),
            jax.ShapeDtypeStruct((N, D), jnp.float32),
        ],
    )(deg2, x)


def _tc_b_body(s0_ref, hp0_ref, dinv_ref, w1_ref, b1_ref, w2_ref, hp2_ref):
    dinv = dinv_ref[...]
    u = dinv * (s0_ref[0] + s0_ref[1] + hp0_ref[...])
    z1 = jnp.maximum(
        jnp.dot(u, w1_ref[...], preferred_element_type=jnp.float32) + b1_ref[...], 0.0)
    h2 = jnp.dot(z1, w2_ref[...], preferred_element_type=jnp.float32)
    hp2_ref[...] = dinv * h2


def _tc_b(s0, hp0, dinv, w1, b1, w2):
    return pl.pallas_call(
        _tc_b_body,
        grid=(_GRID,),
        in_specs=[
            pl.BlockSpec((2, _BR, D), lambda i: (0, i, 0)),
            pl.BlockSpec((_BR, D), lambda i: (i, 0)),
            pl.BlockSpec((_BR, 1), lambda i: (i, 0)),
            pl.BlockSpec((D, 256), lambda i: (0, 0)),
            pl.BlockSpec((1, 256), lambda i: (0, 0)),
            pl.BlockSpec((256, D), lambda i: (0, 0)),
        ],
        out_specs=pl.BlockSpec((_BR, D), lambda i: (i, 0)),
        out_shape=jax.ShapeDtypeStruct((N, D), jnp.float32),
    )(s0, hp0, dinv, w1, b1, w2)


def _tc_c_body(s2_ref, hp2_ref, dinv_ref, b2_ref, w3_ref, hp3_ref):
    dinv = dinv_ref[...]
    z2 = jnp.maximum(dinv * (s2_ref[0] + s2_ref[1] + hp2_ref[...]) + b2_ref[...], 0.0)
    h3 = jnp.dot(z2, w3_ref[...], preferred_element_type=jnp.float32)
    hp3_ref[...] = dinv * h3


def _tc_c(s2, hp2, dinv, b2, w3p):
    return pl.pallas_call(
        _tc_c_body,
        grid=(_GRID,),
        in_specs=[
            pl.BlockSpec((2, _BR, D), lambda i: (0, i, 0)),
            pl.BlockSpec((_BR, D), lambda i: (i, 0)),
            pl.BlockSpec((_BR, 1), lambda i: (i, 0)),
            pl.BlockSpec((1, D), lambda i: (0, 0)),
            pl.BlockSpec((D, 16), lambda i: (0, 0)),
        ],
        out_specs=pl.BlockSpec((_BR, 16), lambda i: (i, 0)),
        out_shape=jax.ShapeDtypeStruct((N, 16), jnp.float32),
    )(s2, hp2, dinv, b2, w3p)


def _tc_d_body(s3_ref, hp3_ref, dinv_ref, b3_ref, out_ref):
    a = dinv_ref[...] * (s3_ref[0] + s3_ref[1] + hp3_ref[...]) + b3_ref[...]
    mask = lax.broadcasted_iota(jnp.int32, (_BR, 16), 1) < 2
    m = jnp.max(jnp.where(mask, a, -jnp.inf), axis=1, keepdims=True)
    s = jnp.sum(jnp.where(mask, jnp.exp(a - m), 0.0), axis=1, keepdims=True)
    out_ref[...] = a - m - jnp.log(s)


def _tc_d(s3, hp3, dinv, b3p):
    return pl.pallas_call(
        _tc_d_body,
        grid=(_GRID,),
        in_specs=[
            pl.BlockSpec((2, _BR, 16), lambda i: (0, i, 0)),
            pl.BlockSpec((_BR, 16), lambda i: (i, 0)),
            pl.BlockSpec((_BR, 1), lambda i: (i, 0)),
            pl.BlockSpec((1, 16), lambda i: (0, 0)),
        ],
        out_specs=pl.BlockSpec((_BR, 16), lambda i: (i, 0)),
        out_shape=jax.ShapeDtypeStruct((N, 16), jnp.float32),
    )(s3, hp3, dinv, b3p)


def kernel(building_x, edge_index, W1, b1, W2, b2, W3, b3):
    src = edge_index[0]
    dst = edge_index[1]
    npad_e = EPAD - E
    pad_src = jnp.arange(npad_e, dtype=jnp.int32) % N
    pad_dst = N + jnp.arange(npad_e, dtype=jnp.int32) % (NPAD - N)
    srcT = jnp.concatenate([src, pad_src]).reshape(NT, NCH, CH)
    dstT = jnp.concatenate([dst, pad_dst]).reshape(NT, NCH, CH)

    b1r = b1.reshape(1, 256)
    b2r = b2.reshape(1, D)
    w3p = jnp.pad(W3, ((0, 0), (0, 16 - W3.shape[1])))
    b3p = jnp.pad(b3, (0, 16 - b3.shape[0])).reshape(1, 16)

    deg2 = _deg_count(dstT)
    dinv, hp0 = _tc_a(deg2, building_x)
    s0 = _edge_agg_128(hp0, srcT, dstT)
    hp2 = _tc_b(s0, hp0, dinv, W1, b1r, W2)
    s2 = _edge_agg_128(hp2, srcT, dstT)
    hp3 = _tc_c(s2, hp2, dinv, b2r, w3p)
    s3 = _edge_agg_16(hp3, srcT, dstT)
    out16 = _tc_d(s3, hp3, dinv, b3p)
    return out16[:, :2]


# speedup vs baseline: 28.6366x; 28.6366x over previous
import functools

import jax
import jax.numpy as jnp
from jax import lax
from jax.experimental import pallas as pl
from jax.experimental.pallas import tpu as pltpu
from jax.experimental.pallas import tpu_sc as plsc

N = 10000
E = 320000
D = 128
NPAD = 10240
NT = 32
CH = 64
NCH = 160
EPAD = NT * NCH * CH
ROWS_PER_TILE = NPAD // 16

_mesh = plsc.VectorSubcoreMesh(core_axis_name="c", subcore_axis_name="s")


def _fill_loop(ref, nrows, width, value):
    def body(i, _):
        for j in range(width // 16):
            ref[i, pl.ds(j * 16, 16)] = jnp.full((16,), value, jnp.float32)
        return 0
    lax.fori_loop(0, nrows, body, 0)


def _make_edge_agg(feat):
    @functools.partial(
        pl.kernel,
        out_type=jax.ShapeDtypeStruct((2, NPAD, feat), jnp.float32),
        mesh=_mesh,
        compiler_params=pltpu.CompilerParams(use_tc_tiling_on_sc=False),
        scratch_types=[
            pltpu.VMEM((NCH, CH), jnp.int32),
            pltpu.VMEM((NCH, CH), jnp.int32),
            pltpu.VMEM((CH, feat), jnp.float32),
            pltpu.VMEM((CH, feat), jnp.float32),
            pltpu.VMEM_SHARED((NPAD, feat), jnp.float32),
            pltpu.SemaphoreType.DMA,
            pltpu.SemaphoreType.DMA,
        ],
    )
    def agg(hp_hbm, srcT_hbm, dstT_hbm, out_hbm, src_v, dst_v, rows0_v, rows1_v,
            acc_sh, sem0, sem1):
        cid = lax.axis_index("c")
        sid = lax.axis_index("s")
        wid = sid * 2 + cid
        pltpu.sync_copy(srcT_hbm.at[wid], src_v)
        pltpu.sync_copy(dstT_hbm.at[wid], dst_v)
        _fill_loop(rows0_v, CH, feat, 0.0)
        for k in range(ROWS_PER_TILE // CH):
            pltpu.sync_copy(rows0_v, acc_sh.at[pl.ds(sid * ROWS_PER_TILE + k * CH, CH)])
        plsc.subcore_barrier()

        pltpu.async_copy(hp_hbm.at[src_v.at[0]], rows0_v, sem0)

        def step(i, _):
            c = 2 * i
            pltpu.async_copy(hp_hbm.at[src_v.at[c + 1]], rows1_v, sem1)
            pltpu.make_async_copy(hp_hbm.at[src_v.at[c]], rows0_v, sem0).wait()
            pltpu.sync_copy(rows0_v, acc_sh.at[dst_v.at[c]], add=True)

            @pl.when(c + 2 < NCH)
            def _():
                pltpu.async_copy(hp_hbm.at[src_v.at[c + 2]], rows0_v, sem0)

            pltpu.make_async_copy(hp_hbm.at[src_v.at[c + 1]], rows1_v, sem1).wait()
            pltpu.sync_copy(rows1_v, acc_sh.at[dst_v.at[c + 1]], add=True)
            return 0
        lax.fori_loop(0, NCH // 2, step, 0)

        plsc.subcore_barrier()
        for k in range(ROWS_PER_TILE // CH):
            sl = pl.ds(sid * ROWS_PER_TILE + k * CH, CH)
            pltpu.sync_copy(acc_sh.at[sl], rows0_v)
            pltpu.sync_copy(rows0_v, out_hbm.at[cid, sl])
    return agg


_edge_agg_128 = _make_edge_agg(128)
_edge_agg_16 = _make_edge_agg(16)


@functools.partial(
    pl.kernel,
    out_type=jax.ShapeDtypeStruct((2, NPAD, 16), jnp.float32),
    mesh=_mesh,
    compiler_params=pltpu.CompilerParams(use_tc_tiling_on_sc=False),
    scratch_types=[
        pltpu.VMEM((NCH, CH), jnp.int32),
        pltpu.VMEM((CH, 16), jnp.float32),
        pltpu.VMEM((CH, 16), jnp.float32),
        pltpu.VMEM_SHARED((NPAD, 16), jnp.float32),
    ],
)
def _deg_count(dstT_hbm, out_hbm, dst_v, ones_v, buf_v, acc_sh):
    cid = lax.axis_index("c")
    sid = lax.axis_index("s")
    wid = sid * 2 + cid
    pltpu.sync_copy(dstT_hbm.at[wid], dst_v)
    _fill_loop(ones_v, CH, 16, 1.0)
    _fill_loop(buf_v, CH, 16, 0.0)
    for k in range(ROWS_PER_TILE // CH):
        pltpu.sync_copy(buf_v, acc_sh.at[pl.ds(sid * ROWS_PER_TILE + k * CH, CH)])
    plsc.subcore_barrier()

    def step(c, _):
        pltpu.sync_copy(ones_v, acc_sh.at[dst_v.at[c]], add=True)
        return 0
    lax.fori_loop(0, NCH, step, 0)

    plsc.subcore_barrier()
    for k in range(ROWS_PER_TILE // CH):
        sl = pl.ds(sid * ROWS_PER_TILE + k * CH, CH)
        pltpu.sync_copy(acc_sh.at[sl], buf_v)
        pltpu.sync_copy(buf_v, out_hbm.at[cid, sl])


_BR = 1000
_GRID = N // _BR


def _tc_a_body(deg_ref, x_ref, dinv_ref, hp_ref):
    deg = deg_ref[0, :, 0] + deg_ref[1, :, 0] + 1.0
    dinv = lax.rsqrt(deg)[:, None]
    dinv_ref[...] = dinv
    hp_ref[...] = x_ref[...] * dinv


def _tc_a(deg2, x):
    return pl.pallas_call(
        _tc_a_body,
        grid=(_GRID,),
        in_specs=[
            pl.BlockSpec((2, _BR, 16), lambda i: (0, i, 0)),
            pl.BlockSpec((_BR, D), lambda i: (i, 0)),
        ],
        out_specs=[
            pl.BlockSpec((_BR, 1), lambda i: (i, 0)),
            pl.BlockSpec((_BR, D), lambda i: (i, 0)),
        ],
        out_shape=[
            jax.ShapeDtypeStruct((N, 1), jnp.float32),
            jax.ShapeDtypeStruct((N, D), jnp.float32),
        ],
    )(deg2, x)


def _tc_b_body(s0_ref, hp0_ref, dinv_ref, w1_ref, b1_ref, w2_ref, hp2_ref):
    dinv = dinv_ref[...]
    u = dinv * (s0_ref[0] + s0_ref[1] + hp0_ref[...])
    z1 = jnp.maximum(
        jnp.dot(u, w1_ref[...], preferred_element_type=jnp.float32) + b1_ref[...], 0.0)
    h2 = jnp.dot(z1, w2_ref[...], preferred_element_type=jnp.float32)
    hp2_ref[...] = dinv * h2


def _tc_b(s0, hp0, dinv, w1, b1, w2):
    return pl.pallas_call(
        _tc_b_body,
        grid=(_GRID,),
        in_specs=[
            pl.BlockSpec((2, _BR, D), lambda i: (0, i, 0)),
            pl.BlockSpec((_BR, D), lambda i: (i, 0)),
            pl.BlockSpec((_BR, 1), lambda i: (i, 0)),
            pl.BlockSpec((D, 256), lambda i: (0, 0)),
            pl.BlockSpec((1, 256), lambda i: (0, 0)),
            pl.BlockSpec((256, D), lambda i: (0, 0)),
        ],
        out_specs=pl.BlockSpec((_BR, D), lambda i: (i, 0)),
        out_shape=jax.ShapeDtypeStruct((N, D), jnp.float32),
    )(s0, hp0, dinv, w1, b1, w2)


def _tc_c_body(s2_ref, hp2_ref, dinv_ref, b2_ref, w3_ref, hp3_ref):
    dinv = dinv_ref[...]
    z2 = jnp.maximum(dinv * (s2_ref[0] + s2_ref[1] + hp2_ref[...]) + b2_ref[...], 0.0)
    h3 = jnp.dot(z2, w3_ref[...], preferred_element_type=jnp.float32)
    hp3_ref[...] = dinv * h3


def _tc_c(s2, hp2, dinv, b2, w3p):
    return pl.pallas_call(
        _tc_c_body,
        grid=(_GRID,),
        in_specs=[
            pl.BlockSpec((2, _BR, D), lambda i: (0, i, 0)),
            pl.BlockSpec((_BR, D), lambda i: (i, 0)),
            pl.BlockSpec((_BR, 1), lambda i: (i, 0)),
            pl.BlockSpec((1, D), lambda i: (0, 0)),
            pl.BlockSpec((D, 16), lambda i: (0, 0)),
        ],
        out_specs=pl.BlockSpec((_BR, 16), lambda i: (i, 0)),
        out_shape=jax.ShapeDtypeStruct((N, 16), jnp.float32),
    )(s2, hp2, dinv, b2, w3p)


def _tc_d_body(s3_ref, hp3_ref, dinv_ref, b3_ref, out_ref):
    a = dinv_ref[...] * (s3_ref[0] + s3_ref[1] + hp3_ref[...]) + b3_ref[...]
    mask = lax.broadcasted_iota(jnp.int32, (_BR, 16), 1) < 2
    m = jnp.max(jnp.where(mask, a, -jnp.inf), axis=1, keepdims=True)
    s = jnp.sum(jnp.where(mask, jnp.exp(a - m), 0.0), axis=1, keepdims=True)
    out_ref[...] = a - m - jnp.log(s)


def _tc_d(s3, hp3, dinv, b3p):
    return pl.pallas_call(
        _tc_d_body,
        grid=(_GRID,),
        in_specs=[
            pl.BlockSpec((2, _BR, 16), lambda i: (0, i, 0)),
            pl.BlockSpec((_BR, 16), lambda i: (i, 0)),
            pl.BlockSpec((_BR, 1), lambda i: (i, 0)),
            pl.BlockSpec((1, 16), lambda i: (0, 0)),
        ],
        out_specs=pl.BlockSpec((_BR, 16), lambda i: (i, 0)),
        out_shape=jax.ShapeDtypeStruct((N, 16), jnp.float32),
    )(s3, hp3, dinv, b3p)


def kernel(building_x, edge_index, W1, b1, W2, b2, W3, b3):
    src = edge_index[0]
    dst = edge_index[1]
    npad_e = EPAD - E
    pad_src = jnp.arange(npad_e, dtype=jnp.int32) % N
    pad_dst = N + jnp.arange(npad_e, dtype=jnp.int32) % (NPAD - N)
    srcT = jnp.concatenate([src, pad_src]).reshape(NT, NCH, CH)
    dstT = jnp.concatenate([dst, pad_dst]).reshape(NT, NCH, CH)

    b1r = b1.reshape(1, 256)
    b2r = b2.reshape(1, D)
    w3p = jnp.pad(W3, ((0, 0), (0, 16 - W3.shape[1])))
    b3p = jnp.pad(b3, (0, 16 - b3.shape[0])).reshape(1, 16)

    deg2 = _deg_count(dstT)
    dinv, hp0 = _tc_a(deg2, building_x)
    s0 = _edge_agg_128(hp0, srcT, dstT)
    hp2 = _tc_b(s0, hp0, dinv, W1, b1r, W2)
    s2 = _edge_agg_128(hp2, srcT, dstT)
    hp3 = _tc_c(s2, hp2, dinv, b2r, w3p)
    s3 = _edge_agg_16(hp3, srcT, dstT)
    out16 = _tc_d(s3, hp3, dinv, b3p)
    return out16[:, :2]


# speedup vs baseline: 30.8236x; 1.0764x over previous
import functools

import jax
import jax.numpy as jnp
from jax import lax
from jax.experimental import pallas as pl
from jax.experimental.pallas import tpu as pltpu
from jax.experimental.pallas import tpu_sc as plsc

N = 10000
E = 320000
D = 128
NPAD = 10240
NT = 32
CH = 64
NCH = 159
EPAD = NT * NCH * CH
ROWS_PER_TILE = NPAD // 16

_mesh = plsc.VectorSubcoreMesh(core_axis_name="c", subcore_axis_name="s")


def _fill_loop(ref, nrows, width, value):
    def body(i, _):
        for j in range(width // 16):
            ref[i, pl.ds(j * 16, 16)] = jnp.full((16,), value, jnp.float32)
        return 0
    lax.fori_loop(0, nrows, body, 0)


def _make_edge_agg(feat):
    @functools.partial(
        pl.kernel,
        out_type=jax.ShapeDtypeStruct((2, NPAD, feat), jnp.float32),
        mesh=_mesh,
        compiler_params=pltpu.CompilerParams(use_tc_tiling_on_sc=False),
        scratch_types=[
            pltpu.VMEM((NCH, CH), jnp.int32),
            pltpu.VMEM((NCH, CH), jnp.int32),
            pltpu.VMEM((CH, feat), jnp.float32),
            pltpu.VMEM((CH, feat), jnp.float32),
            pltpu.VMEM((CH, feat), jnp.float32),
            pltpu.VMEM_SHARED((NPAD, feat), jnp.float32),
            pltpu.SemaphoreType.DMA,
            pltpu.SemaphoreType.DMA,
            pltpu.SemaphoreType.DMA,
            pltpu.SemaphoreType.DMA,
            pltpu.SemaphoreType.DMA,
            pltpu.SemaphoreType.DMA,
        ],
    )
    def agg(hp_hbm, srcT_hbm, dstT_hbm, out_hbm, src_v, dst_v, r0, r1, r2,
            acc_sh, g0, g1, g2, s0, s1, s2):
        rows = (r0, r1, r2)
        gsem = (g0, g1, g2)
        ssem = (s0, s1, s2)
        cid = lax.axis_index("c")
        sid = lax.axis_index("s")
        wid = sid * 2 + cid
        pltpu.sync_copy(srcT_hbm.at[wid], src_v)
        pltpu.sync_copy(dstT_hbm.at[wid], dst_v)
        _fill_loop(r0, CH, feat, 0.0)
        for k in range(ROWS_PER_TILE // CH):
            pltpu.sync_copy(r0, acc_sh.at[pl.ds(sid * ROWS_PER_TILE + k * CH, CH)])
        plsc.subcore_barrier()

        for k in range(3):
            pltpu.async_copy(hp_hbm.at[src_v.at[k]], rows[k], gsem[k])

        def step(i, _):
            c = 3 * i
            for k in range(3):
                pltpu.make_async_copy(hp_hbm.at[src_v.at[c + k]], rows[k], gsem[k]).wait()
                pltpu.async_copy(rows[k], acc_sh.at[dst_v.at[c + k]], ssem[k], add=True)
            for k in range(3):
                pltpu.make_async_copy(rows[k], acc_sh.at[dst_v.at[c + k]], ssem[k]).wait()
                pltpu.async_copy(hp_hbm.at[src_v.at[c + 3 + k]], rows[k], gsem[k])
            return 0
        lax.fori_loop(0, NCH // 3 - 1, step, 0)

        for k in range(3):
            c = NCH - 3 + k
            pltpu.make_async_copy(hp_hbm.at[src_v.at[c]], rows[k], gsem[k]).wait()
            pltpu.sync_copy(rows[k], acc_sh.at[dst_v.at[c]], add=True)

        plsc.subcore_barrier()
        for k in range(ROWS_PER_TILE // CH):
            sl = pl.ds(sid * ROWS_PER_TILE + k * CH, CH)
            pltpu.sync_copy(acc_sh.at[sl], r0)
            pltpu.sync_copy(r0, out_hbm.at[cid, sl])
    return agg


_edge_agg_128 = _make_edge_agg(128)
_edge_agg_16 = _make_edge_agg(16)


@functools.partial(
    pl.kernel,
    out_type=jax.ShapeDtypeStruct((2, NPAD, 16), jnp.float32),
    mesh=_mesh,
    compiler_params=pltpu.CompilerParams(use_tc_tiling_on_sc=False),
    scratch_types=[
        pltpu.VMEM((NCH, CH), jnp.int32),
        pltpu.VMEM((CH, 16), jnp.float32),
        pltpu.VMEM((CH, 16), jnp.float32),
        pltpu.VMEM_SHARED((NPAD, 16), jnp.float32),
        pltpu.SemaphoreType.DMA,
    ],
)
def _deg_count(dstT_hbm, out_hbm, dst_v, ones_v, buf_v, acc_sh, dsem):
    cid = lax.axis_index("c")
    sid = lax.axis_index("s")
    wid = sid * 2 + cid
    pltpu.sync_copy(dstT_hbm.at[wid], dst_v)
    _fill_loop(ones_v, CH, 16, 1.0)
    _fill_loop(buf_v, CH, 16, 0.0)
    for k in range(ROWS_PER_TILE // CH):
        pltpu.sync_copy(buf_v, acc_sh.at[pl.ds(sid * ROWS_PER_TILE + k * CH, CH)])
    plsc.subcore_barrier()

    def step(i, _):
        c = 8 * i
        for k in range(8):
            pltpu.async_copy(ones_v, acc_sh.at[dst_v.at[c + k]], dsem, add=True)
        for k in range(8):
            pltpu.make_async_copy(ones_v, acc_sh.at[dst_v.at[c + k]], dsem).wait()
        return 0
    lax.fori_loop(0, NCH // 8, step, 0)
    for c in range(8 * (NCH // 8), NCH):
        pltpu.sync_copy(ones_v, acc_sh.at[dst_v.at[c]], add=True)

    plsc.subcore_barrier()
    for k in range(ROWS_PER_TILE // CH):
        sl = pl.ds(sid * ROWS_PER_TILE + k * CH, CH)
        pltpu.sync_copy(acc_sh.at[sl], buf_v)
        pltpu.sync_copy(buf_v, out_hbm.at[cid, sl])


_BR = 1000
_GRID = N // _BR


def _tc_a_body(deg_ref, x_ref, dinv_ref, hp_ref):
    deg = deg_ref[0, :, 0] + deg_ref[1, :, 0] + 1.0
    dinv = lax.rsqrt(deg)[:, None]
    dinv_ref[...] = dinv
    hp_ref[...] = x_ref[...] * dinv


def _tc_a(deg2, x):
    return pl.pallas_call(
        _tc_a_body,
        grid=(_GRID,),
        in_specs=[
            pl.BlockSpec((2, _BR, 16), lambda i: (0, i, 0)),
            pl.BlockSpec((_BR, D), lambda i: (i, 0)),
        ],
        out_specs=[
            pl.BlockSpec((_BR, 1), lambda i: (i, 0)),
            pl.BlockSpec((_BR, D), lambda i: (i, 0)),
        ],
        out_shape=[
            jax.ShapeDtypeStruct((N, 1), jnp.float32),
            jax.ShapeDtypeStruct((N, D), jnp.float32),
        ],
    )(deg2, x)


def _tc_b_body(s0_ref, hp0_ref, dinv_ref, w1_ref, b1_ref, w2_ref, hp2_ref):
    dinv = dinv_ref[...]
    u = dinv * (s0_ref[0] + s0_ref[1] + hp0_ref[...])
    z1 = jnp.maximum(
        jnp.dot(u, w1_ref[...], preferred_element_type=jnp.float32) + b1_ref[...], 0.0)
    h2 = jnp.dot(z1, w2_ref[...], preferred_element_type=jnp.float32)
    hp2_ref[...] = dinv * h2


def _tc_b(s0, hp0, dinv, w1, b1, w2):
    return pl.pallas_call(
        _tc_b_body,
        grid=(_GRID,),
        in_specs=[
            pl.BlockSpec((2, _BR, D), lambda i: (0, i, 0)),
            pl.BlockSpec((_BR, D), lambda i: (i, 0)),
            pl.BlockSpec((_BR, 1), lambda i: (i, 0)),
            pl.BlockSpec((D, 256), lambda i: (0, 0)),
            pl.BlockSpec((1, 256), lambda i: (0, 0)),
            pl.BlockSpec((256, D), lambda i: (0, 0)),
        ],
        out_specs=pl.BlockSpec((_BR, D), lambda i: (i, 0)),
        out_shape=jax.ShapeDtypeStruct((N, D), jnp.float32),
    )(s0, hp0, dinv, w1, b1, w2)


def _tc_c_body(s2_ref, hp2_ref, dinv_ref, b2_ref, w3_ref, hp3_ref):
    dinv = dinv_ref[...]
    z2 = jnp.maximum(dinv * (s2_ref[0] + s2_ref[1] + hp2_ref[...]) + b2_ref[...], 0.0)
    h3 = jnp.dot(z2, w3_ref[...], preferred_element_type=jnp.float32)
    hp3_ref[...] = dinv * h3


def _tc_c(s2, hp2, dinv, b2, w3p):
    return pl.pallas_call(
        _tc_c_body,
        grid=(_GRID,),
        in_specs=[
            pl.BlockSpec((2, _BR, D), lambda i: (0, i, 0)),
            pl.BlockSpec((_BR, D), lambda i: (i, 0)),
            pl.BlockSpec((_BR, 1), lambda i: (i, 0)),
            pl.BlockSpec((1, D), lambda i: (0, 0)),
            pl.BlockSpec((D, 16), lambda i: (0, 0)),
        ],
        out_specs=pl.BlockSpec((_BR, 16), lambda i: (i, 0)),
        out_shape=jax.ShapeDtypeStruct((N, 16), jnp.float32),
    )(s2, hp2, dinv, b2, w3p)


def _tc_d_body(s3_ref, hp3_ref, dinv_ref, b3_ref, out_ref):
    a = dinv_ref[...] * (s3_ref[0] + s3_ref[1] + hp3_ref[...]) + b3_ref[...]
    mask = lax.broadcasted_iota(jnp.int32, (_BR, 16), 1) < 2
    m = jnp.max(jnp.where(mask, a, -jnp.inf), axis=1, keepdims=True)
    s = jnp.sum(jnp.where(mask, jnp.exp(a - m), 0.0), axis=1, keepdims=True)
    out_ref[...] = a - m - jnp.log(s)


def _tc_d(s3, hp3, dinv, b3p):
    return pl.pallas_call(
        _tc_d_body,
        grid=(_GRID,),
        in_specs=[
            pl.BlockSpec((2, _BR, 16), lambda i: (0, i, 0)),
            pl.BlockSpec((_BR, 16), lambda i: (i, 0)),
            pl.BlockSpec((_BR, 1), lambda i: (i, 0)),
            pl.BlockSpec((1, 16), lambda i: (0, 0)),
        ],
        out_specs=pl.BlockSpec((_BR, 16), lambda i: (i, 0)),
        out_shape=jax.ShapeDtypeStruct((N, 16), jnp.float32),
    )(s3, hp3, dinv, b3p)


def kernel(building_x, edge_index, W1, b1, W2, b2, W3, b3):
    src = edge_index[0]
    dst = edge_index[1]
    npad_e = EPAD - E
    pad_src = jnp.arange(npad_e, dtype=jnp.int32) % N
    pad_dst = N + jnp.arange(npad_e, dtype=jnp.int32) % (NPAD - N)
    srcT = jnp.concatenate([src, pad_src]).reshape(NT, NCH, CH)
    dstT = jnp.concatenate([dst, pad_dst]).reshape(NT, NCH, CH)

    b1r = b1.reshape(1, 256)
    b2r = b2.reshape(1, D)
    w3p = jnp.pad(W3, ((0, 0), (0, 16 - W3.shape[1])))
    b3p = jnp.pad(b3, (0, 16 - b3.shape[0])).reshape(1, 16)

    deg2 = _deg_count(dstT)
    dinv, hp0 = _tc_a(deg2, building_x)
    s0 = _edge_agg_128(hp0, srcT, dstT)
    hp2 = _tc_b(s0, hp0, dinv, W1, b1r, W2)
    s2 = _edge_agg_128(hp2, srcT, dstT)
    hp3 = _tc_c(s2, hp2, dinv, b2r, w3p)
    s3 = _edge_agg_16(hp3, srcT, dstT)
    out16 = _tc_d(s3, hp3, dinv, b3p)
    return out16[:, :2]


# speedup vs baseline: 32.4726x; 1.0535x over previous
import functools

import jax
import jax.numpy as jnp
from jax import lax
from jax.experimental import pallas as pl
from jax.experimental.pallas import tpu as pltpu
from jax.experimental.pallas import tpu_sc as plsc

N = 10000
E = 320000
D = 128
NPAD = 10240
NT = 32
CH = 64
NCH = 159
EPAD = NT * NCH * CH
CH2 = 128
NCH2 = 81
EPAD2 = NT * NCH2 * CH2
ROWS_PER_TILE = NPAD // 16

_mesh = plsc.VectorSubcoreMesh(core_axis_name="c", subcore_axis_name="s")


def _fill_loop(ref, nrows, width, value):
    def body(i, _):
        for j in range(width // 16):
            ref[i, pl.ds(j * 16, 16)] = jnp.full((16,), value, jnp.float32)
        return 0
    lax.fori_loop(0, nrows, body, 0)


def _make_edge_agg(feat, ch, nch):
    @functools.partial(
        pl.kernel,
        out_type=jax.ShapeDtypeStruct((2, NPAD, feat), jnp.float32),
        mesh=_mesh,
        compiler_params=pltpu.CompilerParams(use_tc_tiling_on_sc=False),
        scratch_types=[
            pltpu.VMEM((nch, ch), jnp.int32),
            pltpu.VMEM((nch, ch), jnp.int32),
            pltpu.VMEM((ch, feat), jnp.float32),
            pltpu.VMEM((ch, feat), jnp.float32),
            pltpu.VMEM((ch, feat), jnp.float32),
            pltpu.VMEM_SHARED((NPAD, feat), jnp.float32),
            pltpu.SemaphoreType.DMA,
            pltpu.SemaphoreType.DMA,
            pltpu.SemaphoreType.DMA,
            pltpu.SemaphoreType.DMA,
            pltpu.SemaphoreType.DMA,
            pltpu.SemaphoreType.DMA,
        ],
    )
    def agg(hp_hbm, srcT_hbm, dstT_hbm, out_hbm, src_v, dst_v, r0, r1, r2,
            acc_sh, g0, g1, g2, s0, s1, s2):
        rows = (r0, r1, r2)
        gsem = (g0, g1, g2)
        ssem = (s0, s1, s2)
        cid = lax.axis_index("c")
        sid = lax.axis_index("s")
        wid = sid * 2 + cid
        pltpu.sync_copy(srcT_hbm.at[wid], src_v)
        pltpu.sync_copy(dstT_hbm.at[wid], dst_v)
        _fill_loop(r0, ch, feat, 0.0)
        for k in range(ROWS_PER_TILE // ch):
            pltpu.sync_copy(r0, acc_sh.at[pl.ds(sid * ROWS_PER_TILE + k * ch, ch)])
        plsc.subcore_barrier()

        for k in range(3):
            pltpu.async_copy(hp_hbm.at[src_v.at[k]], rows[k], gsem[k])

        def step(i, _):
            c = 3 * i
            for k in range(3):
                pltpu.make_async_copy(hp_hbm.at[src_v.at[c + k]], rows[k], gsem[k]).wait()
                pltpu.async_copy(rows[k], acc_sh.at[dst_v.at[c + k]], ssem[k], add=True)
            for k in range(3):
                pltpu.make_async_copy(rows[k], acc_sh.at[dst_v.at[c + k]], ssem[k]).wait()
                pltpu.async_copy(hp_hbm.at[src_v.at[c + 3 + k]], rows[k], gsem[k])
            return 0
        lax.fori_loop(0, nch // 3 - 1, step, 0)

        for k in range(3):
            c = nch - 3 + k
            pltpu.make_async_copy(hp_hbm.at[src_v.at[c]], rows[k], gsem[k]).wait()
            pltpu.sync_copy(rows[k], acc_sh.at[dst_v.at[c]], add=True)

        plsc.subcore_barrier()
        for k in range(ROWS_PER_TILE // ch):
            sl = pl.ds(sid * ROWS_PER_TILE + k * ch, ch)
            pltpu.sync_copy(acc_sh.at[sl], r0)
            pltpu.sync_copy(r0, out_hbm.at[cid, sl])
    return agg


_edge_agg_128 = _make_edge_agg(128, CH, NCH)
_edge_agg_16 = _make_edge_agg(16, CH2, NCH2)


@functools.partial(
    pl.kernel,
    out_type=jax.ShapeDtypeStruct((2, NPAD, 16), jnp.float32),
    mesh=_mesh,
    compiler_params=pltpu.CompilerParams(use_tc_tiling_on_sc=False),
    scratch_types=[
        pltpu.VMEM((NCH2, CH2), jnp.int32),
        pltpu.VMEM((CH2, 16), jnp.float32),
        pltpu.VMEM((CH2, 16), jnp.float32),
        pltpu.VMEM_SHARED((NPAD, 16), jnp.float32),
        pltpu.SemaphoreType.DMA,
    ],
)
def _deg_count(dstT_hbm, out_hbm, dst_v, ones_v, buf_v, acc_sh, dsem):
    cid = lax.axis_index("c")
    sid = lax.axis_index("s")
    wid = sid * 2 + cid
    pltpu.sync_copy(dstT_hbm.at[wid], dst_v)
    _fill_loop(ones_v, CH2, 16, 1.0)
    _fill_loop(buf_v, CH2, 16, 0.0)
    for k in range(ROWS_PER_TILE // CH2):
        pltpu.sync_copy(buf_v, acc_sh.at[pl.ds(sid * ROWS_PER_TILE + k * CH2, CH2)])
    plsc.subcore_barrier()

    def step(i, _):
        c = 8 * i
        for k in range(8):
            pltpu.async_copy(ones_v, acc_sh.at[dst_v.at[c + k]], dsem, add=True)
        for k in range(8):
            pltpu.make_async_copy(ones_v, acc_sh.at[dst_v.at[c + k]], dsem).wait()
        return 0
    lax.fori_loop(0, NCH2 // 8, step, 0)
    for c in range(8 * (NCH2 // 8), NCH2):
        pltpu.sync_copy(ones_v, acc_sh.at[dst_v.at[c]], add=True)

    plsc.subcore_barrier()
    for k in range(ROWS_PER_TILE // CH2):
        sl = pl.ds(sid * ROWS_PER_TILE + k * CH2, CH2)
        pltpu.sync_copy(acc_sh.at[sl], buf_v)
        pltpu.sync_copy(buf_v, out_hbm.at[cid, sl])


_BR = 1000
_GRID = N // _BR


def _tc_a_body(deg_ref, x_ref, dinv_ref, hp_ref):
    deg = deg_ref[0, :, 0] + deg_ref[1, :, 0] + 1.0
    dinv = lax.rsqrt(deg)[:, None]
    dinv_ref[...] = dinv
    hp_ref[...] = x_ref[...] * dinv


def _tc_a(deg2, x):
    return pl.pallas_call(
        _tc_a_body,
        grid=(_GRID,),
        in_specs=[
            pl.BlockSpec((2, _BR, 16), lambda i: (0, i, 0)),
            pl.BlockSpec((_BR, D), lambda i: (i, 0)),
        ],
        out_specs=[
            pl.BlockSpec((_BR, 1), lambda i: (i, 0)),
            pl.BlockSpec((_BR, D), lambda i: (i, 0)),
        ],
        out_shape=[
            jax.ShapeDtypeStruct((N, 1), jnp.float32),
            jax.ShapeDtypeStruct((N, D), jnp.float32),
        ],
    )(deg2, x)


def _tc_b_body(s0_ref, hp0_ref, dinv_ref, w1_ref, b1_ref, w2_ref, hp2_ref):
    dinv = dinv_ref[...]
    u = dinv * (s0_ref[0] + s0_ref[1] + hp0_ref[...])
    z1 = jnp.maximum(
        jnp.dot(u, w1_ref[...], preferred_element_type=jnp.float32) + b1_ref[...], 0.0)
    h2 = jnp.dot(z1, w2_ref[...], preferred_element_type=jnp.float32)
    hp2_ref[...] = dinv * h2


def _tc_b(s0, hp0, dinv, w1, b1, w2):
    return pl.pallas_call(
        _tc_b_body,
        grid=(_GRID,),
        in_specs=[
            pl.BlockSpec((2, _BR, D), lambda i: (0, i, 0)),
            pl.BlockSpec((_BR, D), lambda i: (i, 0)),
            pl.BlockSpec((_BR, 1), lambda i: (i, 0)),
            pl.BlockSpec((D, 256), lambda i: (0, 0)),
            pl.BlockSpec((1, 256), lambda i: (0, 0)),
            pl.BlockSpec((256, D), lambda i: (0, 0)),
        ],
        out_specs=pl.BlockSpec((_BR, D), lambda i: (i, 0)),
        out_shape=jax.ShapeDtypeStruct((N, D), jnp.float32),
    )(s0, hp0, dinv, w1, b1, w2)


def _tc_c_body(s2_ref, hp2_ref, dinv_ref, b2_ref, w3_ref, hp3_ref):
    dinv = dinv_ref[...]
    z2 = jnp.maximum(dinv * (s2_ref[0] + s2_ref[1] + hp2_ref[...]) + b2_ref[...], 0.0)
    h3 = jnp.dot(z2, w3_ref[...], preferred_element_type=jnp.float32)
    hp3_ref[...] = dinv * h3


def _tc_c(s2, hp2, dinv, b2, w3p):
    return pl.pallas_call(
        _tc_c_body,
        grid=(_GRID,),
        in_specs=[
            pl.BlockSpec((2, _BR, D), lambda i: (0, i, 0)),
            pl.BlockSpec((_BR, D), lambda i: (i, 0)),
            pl.BlockSpec((_BR, 1), lambda i: (i, 0)),
            pl.BlockSpec((1, D), lambda i: (0, 0)),
            pl.BlockSpec((D, 16), lambda i: (0, 0)),
        ],
        out_specs=pl.BlockSpec((_BR, 16), lambda i: (i, 0)),
        out_shape=jax.ShapeDtypeStruct((N, 16), jnp.float32),
    )(s2, hp2, dinv, b2, w3p)


def _tc_d_body(s3_ref, hp3_ref, dinv_ref, b3_ref, out_ref):
    a = dinv_ref[...] * (s3_ref[0] + s3_ref[1] + hp3_ref[...]) + b3_ref[...]
    mask = lax.broadcasted_iota(jnp.int32, (_BR, 16), 1) < 2
    m = jnp.max(jnp.where(mask, a, -jnp.inf), axis=1, keepdims=True)
    s = jnp.sum(jnp.where(mask, jnp.exp(a - m), 0.0), axis=1, keepdims=True)
    out_ref[...] = a - m - jnp.log(s)


def _tc_d(s3, hp3, dinv, b3p):
    return pl.pallas_call(
        _tc_d_body,
        grid=(_GRID,),
        in_specs=[
            pl.BlockSpec((2, _BR, 16), lambda i: (0, i, 0)),
            pl.BlockSpec((_BR, 16), lambda i: (i, 0)),
            pl.BlockSpec((_BR, 1), lambda i: (i, 0)),
            pl.BlockSpec((1, 16), lambda i: (0, 0)),
        ],
        out_specs=pl.BlockSpec((_BR, 16), lambda i: (i, 0)),
        out_shape=jax.ShapeDtypeStruct((N, 16), jnp.float32),
    )(s3, hp3, dinv, b3p)


def kernel(building_x, edge_index, W1, b1, W2, b2, W3, b3):
    src = edge_index[0]
    dst = edge_index[1]

    def strips(n_pad_e, nch, ch):
        pad_src = jnp.arange(n_pad_e, dtype=jnp.int32) % N
        pad_dst = N + jnp.arange(n_pad_e, dtype=jnp.int32) % (NPAD - N)
        return (jnp.concatenate([src, pad_src]).reshape(NT, nch, ch),
                jnp.concatenate([dst, pad_dst]).reshape(NT, nch, ch))

    srcT, dstT = strips(EPAD - E, NCH, CH)
    srcT2, dstT2 = strips(EPAD2 - E, NCH2, CH2)

    b1r = b1.reshape(1, 256)
    b2r = b2.reshape(1, D)
    w3p = jnp.pad(W3, ((0, 0), (0, 16 - W3.shape[1])))
    b3p = jnp.pad(b3, (0, 16 - b3.shape[0])).reshape(1, 16)

    deg2 = _deg_count(dstT2)
    dinv, hp0 = _tc_a(deg2, building_x)
    s0 = _edge_agg_128(hp0, srcT, dstT)
    hp2 = _tc_b(s0, hp0, dinv, W1, b1r, W2)
    s2 = _edge_agg_128(hp2, srcT, dstT)
    hp3 = _tc_c(s2, hp2, dinv, b2r, w3p)
    s3 = _edge_agg_16(hp3, srcT2, dstT2)
    out16 = _tc_d(s3, hp3, dinv, b3p)
    return out16[:, :2]
